# speedup vs baseline: 3.6730x; 3.6730x over previous
import functools

import jax
import jax.numpy as jnp
from jax import lax
from jax.experimental import pallas as pl
from jax.experimental.pallas import tpu as pltpu
from jax.experimental.pallas import tpu_sc as plsc

_D = 128
_B = 4096
_H = 50
_TOT = _B * _H
_NW = 32
_PER_W = _TOT // _NW
_CHUNK = 128
_NCH = _PER_W // _CHUNK


def _emb_body(text_hbm, wf_hbm, wt_hbm, out_hbm,
              idx_v, buf_f, buf_t, sem_f, sem_t):
    wid = lax.axis_index("s") * 2 + lax.axis_index("c")
    base = wid * _PER_W
    pltpu.sync_copy(text_hbm.at[pl.ds(base, _PER_W)], idx_v)

    def step(c, carry):
        row0 = c * _CHUNK
        idxc = idx_v.at[pl.ds(row0, _CHUNK)]
        cf = pltpu.async_copy(wf_hbm.at[idxc], buf_f, sem_f)
        ct = pltpu.async_copy(wt_hbm.at[idxc], buf_t, sem_t)
        cf.wait()
        ct.wait()
        pltpu.sync_copy(buf_f, out_hbm.at[pl.ds(base + row0, _CHUNK),
                                          pl.ds(0, _D)])
        pltpu.sync_copy(buf_t, out_hbm.at[pl.ds(base + row0, _CHUNK),
                                          pl.ds(_D, _D)])
        return carry

    lax.fori_loop(0, _NCH, step, 0)


@functools.partial(
    pl.kernel,
    out_type=jax.ShapeDtypeStruct((_TOT, 2 * _D), jnp.float32),
    mesh=plsc.VectorSubcoreMesh(core_axis_name="c", subcore_axis_name="s"),
    scratch_types=[
        pltpu.VMEM((_PER_W,), jnp.int32),
        pltpu.VMEM((_CHUNK, _D), jnp.float32),
        pltpu.VMEM((_CHUNK, _D), jnp.float32),
        pltpu.SemaphoreType.DMA,
        pltpu.SemaphoreType.DMA,
    ],
)
def _emb_lookup(text_hbm, wf_hbm, wt_hbm, out_hbm,
                idx_v, buf_f, buf_t, sem_f, sem_t):
    _emb_body(text_hbm, wf_hbm, wt_hbm, out_hbm,
              idx_v, buf_f, buf_t, sem_f, sem_t)


def kernel(text, W_frozen, W_train):
    flat = text.reshape(_TOT).astype(jnp.int32)
    out = _emb_lookup(flat, W_frozen, W_train)
    return out.reshape(_B, _H, 2 * _D)


# speedup vs baseline: 3.8553x; 1.0496x over previous
import functools

import jax
import jax.numpy as jnp
from jax import lax
from jax.experimental import pallas as pl
from jax.experimental.pallas import tpu as pltpu
from jax.experimental.pallas import tpu_sc as plsc

_D = 128
_B = 4096
_H = 50
_TOT = _B * _H
_NW = 32
_PER_W = _TOT // _NW
_CHUNK = 128
_NCH = _PER_W // _CHUNK


def _emb_body(text_hbm, wf_hbm, wt_hbm, out_hbm,
              idx_v, buf_f, buf_t, sem_f, sem_t):
    wid = lax.axis_index("s") * 2 + lax.axis_index("c")
    base = wid * _PER_W
    pltpu.sync_copy(text_hbm.at[pl.ds(base, _PER_W)], idx_v)

    def issue(c, b):
        idxc = idx_v.at[pl.ds(c * _CHUNK, _CHUNK)]
        pltpu.async_copy(wf_hbm.at[idxc], buf_f.at[b], sem_f.at[b])
        pltpu.async_copy(wt_hbm.at[idxc], buf_t.at[b], sem_t.at[b])

    def wait_gather(c, b):
        idxc = idx_v.at[pl.ds(c * _CHUNK, _CHUNK)]
        pltpu.make_async_copy(wf_hbm.at[idxc], buf_f.at[b], sem_f.at[b]).wait()
        pltpu.make_async_copy(wt_hbm.at[idxc], buf_t.at[b], sem_t.at[b]).wait()

    issue(0, 0)

    @pl.loop(0, _NCH, step=2)
    def _chunk_loop(c0):
        for b in range(2):
            c = c0 + b

            @pl.when(c + 1 < _NCH)
            def _():
                issue(c + 1, 1 - b)

            wait_gather(c, b)
            row0 = base + c * _CHUNK
            pltpu.sync_copy(buf_f.at[b],
                            out_hbm.at[pl.ds(row0, _CHUNK), pl.ds(0, _D)])
            pltpu.sync_copy(buf_t.at[b],
                            out_hbm.at[pl.ds(row0, _CHUNK), pl.ds(_D, _D)])


@functools.partial(
    pl.kernel,
    out_type=jax.ShapeDtypeStruct((_TOT, 2 * _D), jnp.float32),
    mesh=plsc.VectorSubcoreMesh(core_axis_name="c", subcore_axis_name="s"),
    scratch_types=[
        pltpu.VMEM((_PER_W,), jnp.int32),
        pltpu.VMEM((2, _CHUNK, _D), jnp.float32),
        pltpu.VMEM((2, _CHUNK, _D), jnp.float32),
        pltpu.SemaphoreType.DMA((2,)),
        pltpu.SemaphoreType.DMA((2,)),
    ],
)
def _emb_lookup(text_hbm, wf_hbm, wt_hbm, out_hbm,
                idx_v, buf_f, buf_t, sem_f, sem_t):
    _emb_body(text_hbm, wf_hbm, wt_hbm, out_hbm,
              idx_v, buf_f, buf_t, sem_f, sem_t)


def kernel(text, W_frozen, W_train):
    flat = text.reshape(_TOT).astype(jnp.int32)
    out = _emb_lookup(flat, W_frozen, W_train)
    return out.reshape(_B, _H, 2 * _D)


# speedup vs baseline: 3.8657x; 1.0027x over previous
import functools

import jax
import jax.numpy as jnp
from jax import lax
from jax.experimental import pallas as pl
from jax.experimental.pallas import tpu as pltpu
from jax.experimental.pallas import tpu_sc as plsc

_D = 128
_B = 4096
_H = 50
_TOT = _B * _H
_NW = 32
_PER_W = _TOT // _NW
_CHUNK = 128
_NCH = _PER_W // _CHUNK


def _emb_body(text_hbm, wf_hbm, wt_hbm, out_hbm,
              idx_v, buf, sem_f, sem_t):
    wid = lax.axis_index("s") * 2 + lax.axis_index("c")
    base = wid * _PER_W
    pltpu.sync_copy(text_hbm.at[pl.ds(base, _PER_W)], idx_v)

    def issue(c, b):
        idxc = idx_v.at[pl.ds(c * _CHUNK, _CHUNK)]
        pltpu.async_copy(wf_hbm.at[idxc], buf.at[b, :, pl.ds(0, _D)],
                         sem_f.at[b])
        pltpu.async_copy(wt_hbm.at[idxc], buf.at[b, :, pl.ds(_D, _D)],
                         sem_t.at[b])

    def wait_gather(c, b):
        idxc = idx_v.at[pl.ds(c * _CHUNK, _CHUNK)]
        pltpu.make_async_copy(wf_hbm.at[idxc], buf.at[b, :, pl.ds(0, _D)],
                              sem_f.at[b]).wait()
        pltpu.make_async_copy(wt_hbm.at[idxc], buf.at[b, :, pl.ds(_D, _D)],
                              sem_t.at[b]).wait()

    issue(0, 0)

    @pl.loop(0, _NCH, step=2)
    def _chunk_loop(c0):
        for b in range(2):
            c = c0 + b

            @pl.when(c + 1 < _NCH)
            def _():
                issue(c + 1, 1 - b)

            wait_gather(c, b)
            row0 = base + c * _CHUNK
            pltpu.sync_copy(buf.at[b], out_hbm.at[pl.ds(row0, _CHUNK), :])


@functools.partial(
    pl.kernel,
    out_type=jax.ShapeDtypeStruct((_TOT, 2 * _D), jnp.float32),
    mesh=plsc.VectorSubcoreMesh(core_axis_name="c", subcore_axis_name="s"),
    scratch_types=[
        pltpu.VMEM((_PER_W,), jnp.int32),
        pltpu.VMEM((2, _CHUNK, 2 * _D), jnp.float32),
        pltpu.SemaphoreType.DMA((2,)),
        pltpu.SemaphoreType.DMA((2,)),
    ],
)
def _emb_lookup(text_hbm, wf_hbm, wt_hbm, out_hbm,
                idx_v, buf, sem_f, sem_t):
    _emb_body(text_hbm, wf_hbm, wt_hbm, out_hbm,
              idx_v, buf, sem_f, sem_t)


def kernel(text, W_frozen, W_train):
    flat = text.reshape(_TOT).astype(jnp.int32)
    out = _emb_lookup(flat, W_frozen, W_train)
    return out.reshape(_B, _H, 2 * _D)


# speedup vs baseline: 12.5409x; 3.2442x over previous
import functools

import jax
import jax.numpy as jnp
from jax import lax
from jax.experimental import pallas as pl
from jax.experimental.pallas import tpu as pltpu
from jax.experimental.pallas import tpu_sc as plsc

_D = 128
_B = 4096
_H = 50
_NW = 32
_BC = _B // _NW


def _emb_body(textt_hbm, wf_hbm, wt_hbm, out_hbm, idx_v, buf, sem_f, sem_t):
    wid = lax.axis_index("s") * 2 + lax.axis_index("c")
    b0 = wid * _BC
    pltpu.sync_copy(textt_hbm.at[:, pl.ds(b0, _BC)], idx_v)

    def issue(h, b):
        idxc = idx_v.at[h]
        pltpu.async_copy(wf_hbm.at[idxc], buf.at[b, :, pl.ds(0, _D)],
                         sem_f.at[b])
        pltpu.async_copy(wt_hbm.at[idxc], buf.at[b, :, pl.ds(_D, _D)],
                         sem_t.at[b])

    def wait_gather(h, b):
        idxc = idx_v.at[h]
        pltpu.make_async_copy(wf_hbm.at[idxc], buf.at[b, :, pl.ds(0, _D)],
                              sem_f.at[b]).wait()
        pltpu.make_async_copy(wt_hbm.at[idxc], buf.at[b, :, pl.ds(_D, _D)],
                              sem_t.at[b]).wait()

    issue(0, 0)

    @pl.loop(0, _H, step=2)
    def _chunk_loop(h0):
        for b in range(2):
            h = h0 + b

            @pl.when(h + 1 < _H)
            def _():
                issue(h + 1, 1 - b)

            wait_gather(h, b)
            pltpu.sync_copy(buf.at[b], out_hbm.at[h, pl.ds(b0, _BC), :])


@functools.partial(
    pl.kernel,
    out_type=jax.ShapeDtypeStruct((_H, _B, 2 * _D), jnp.float32),
    mesh=plsc.VectorSubcoreMesh(core_axis_name="c", subcore_axis_name="s"),
    scratch_types=[
        pltpu.VMEM((_H, _BC), jnp.int32),
        pltpu.VMEM((2, _BC, 2 * _D), jnp.float32),
        pltpu.SemaphoreType.DMA((2,)),
        pltpu.SemaphoreType.DMA((2,)),
    ],
)
def _emb_lookup(textt_hbm, wf_hbm, wt_hbm, out_hbm, idx_v, buf, sem_f, sem_t):
    _emb_body(textt_hbm, wf_hbm, wt_hbm, out_hbm, idx_v, buf, sem_f, sem_t)


def kernel(text, W_frozen, W_train):
    textt = text.T.astype(jnp.int32)
    out = _emb_lookup(textt, W_frozen, W_train)
    return out.transpose(1, 0, 2)
